# Initial kernel scaffold; baseline (speedup 1.0000x reference)
import functools

import jax
import jax.numpy as jnp
from jax import lax
from jax.experimental import pallas as pl
from jax.experimental.pallas import tpu as pltpu
from jax.experimental.pallas import tpu_sc as plsc

N = 10000
D = 256
R = 4
E = 40000

NC = 2
NS = 16
BLK = 128
EPAD = 40960
NBLK = EPAD // (NS * BLK)
DP = 288
DH = DP // NC
DEGCOL = 256 % DH
NROW = N + 8
RPS = N // NS
ZROWS = 125

_sc_mesh = plsc.VectorSubcoreMesh(
    core_axis_name="c", subcore_axis_name="s", num_cores=NC, num_subcores=NS)


def _make_sc_kernel():
  @functools.partial(
      pl.kernel,
      out_type=jax.ShapeDtypeStruct((NC, R, N, DH), jnp.float32),
      mesh=_sc_mesh,
      scratch_types=[
          pltpu.VMEM((ZROWS, DH), jnp.float32),
          pltpu.VMEM((NBLK, BLK), jnp.int32),
          pltpu.VMEM((NBLK, BLK), jnp.int32),
          pltpu.VMEM((2, BLK, DH), jnp.float32),
          pltpu.VMEM_SHARED((NROW, DH), jnp.float32),
          pltpu.SemaphoreType.DMA,
          pltpu.SemaphoreType.DMA,
      ],
  )
  def sc_agg(xs_hbm, ei_hbm, z_hbm, agg_hbm,
             zbuf, srcb, dstb, rowb, shared, sem0, sem1):
    c = lax.axis_index("c")
    s = lax.axis_index("s")
    sems = (sem0, sem1)

    pltpu.sync_copy(z_hbm, zbuf)

    @pl.when(s == 0)
    def _():
      pltpu.sync_copy(zbuf.at[pl.ds(0, NROW - N)],
                      shared.at[pl.ds(N, NROW - N)])

    for r in range(R):
      for z in range(RPS // ZROWS):
        pltpu.sync_copy(zbuf, shared.at[pl.ds(s * RPS + z * ZROWS, ZROWS)])
      plsc.subcore_barrier()

      pltpu.sync_copy(ei_hbm.at[r, 0, s], srcb)
      pltpu.sync_copy(ei_hbm.at[r, 1, s], dstb)

      d = pltpu.async_copy(xs_hbm.at[c].at[srcb.at[0]], rowb.at[0], sems[0])
      for j in range(NBLK):
        nxt = None
        if j + 1 < NBLK:
          nxt = pltpu.async_copy(
              xs_hbm.at[c].at[srcb.at[j + 1]],
              rowb.at[(j + 1) % 2], sems[(j + 1) % 2])
        d.wait()
        pltpu.sync_copy(rowb.at[j % 2], shared.at[dstb.at[j]], add=True)
        d = nxt
      plsc.subcore_barrier()

      for z in range(RPS // ZROWS):
        rowlo = s * RPS + z * ZROWS
        pltpu.sync_copy(shared.at[pl.ds(rowlo, ZROWS)],
                        agg_hbm.at[c, r, pl.ds(rowlo, ZROWS)])

  return sc_agg


_sc_agg = _make_sc_kernel()

MB = 2000


def _tc_body(x_ref, agg_ref, w2_ref, lw_ref, o_ref):
  acc = jnp.dot(x_ref[...], lw_ref[...], preferred_element_type=jnp.float32)
  for r in range(R):
    a0 = agg_ref[0, r]
    a1 = agg_ref[1, r]
    rec = 1.0 / jnp.maximum(a1[:, DEGCOL:DEGCOL + 1], 1.0)
    acc = acc + jnp.dot(a0 * rec, w2_ref[0, r],
                        preferred_element_type=jnp.float32)
    acc = acc + jnp.dot(a1 * rec, w2_ref[1, r],
                        preferred_element_type=jnp.float32)
  o_ref[...] = acc


def _tc_matmul(x, agg, w2, lw):
  return pl.pallas_call(
      _tc_body,
      grid=(N // MB,),
      in_specs=[
          pl.BlockSpec((MB, D), lambda i: (i, 0)),
          pl.BlockSpec((NC, R, MB, DH), lambda i: (0, 0, i, 0)),
          pl.BlockSpec((NC, R, DH, D), lambda i: (0, 0, 0, 0)),
          pl.BlockSpec((D, D), lambda i: (0, 0)),
      ],
      out_specs=pl.BlockSpec((MB, D), lambda i: (i, 0)),
      out_shape=jax.ShapeDtypeStruct((N, D), jnp.float32),
  )(x, agg, w2, lw)


def kernel(x, edge_index_r0, edge_index_r1, edge_index_r2, edge_index_r3,
           weight, loop_weight):
  xs = jnp.concatenate(
      [x, jnp.ones((N, 1), jnp.float32),
       jnp.zeros((N, DP - D - 1), jnp.float32)], axis=1)
  xs = xs.reshape(N, NC, DH).transpose(1, 0, 2)

  pad = EPAD - E
  eis = []
  for ei in (edge_index_r0, edge_index_r1, edge_index_r2, edge_index_r3):
    eis.append(jnp.concatenate(
        [ei, jnp.concatenate([jnp.zeros((1, pad), jnp.int32),
                              jnp.full((1, pad), N, jnp.int32)], axis=0)],
        axis=1))
  ei = jnp.stack(eis).reshape(R, 2, NS, NBLK, BLK)

  zeros = jnp.zeros((ZROWS, DH), jnp.float32)
  agg = _sc_agg(xs, ei, zeros)

  wpad = jnp.concatenate(
      [weight, jnp.zeros((R, DP - D, D), jnp.float32)], axis=1)
  w2 = wpad.reshape(R, NC, DH, D).transpose(1, 0, 2, 3)

  return _tc_matmul(x, agg, w2, loop_weight)


# speedup vs baseline: 3.0977x; 3.0977x over previous
import functools

import jax
import jax.numpy as jnp
from jax import lax
from jax.experimental import pallas as pl
from jax.experimental.pallas import tpu as pltpu
from jax.experimental.pallas import tpu_sc as plsc

N = 10000
D = 256
R = 4
E = 40000

NC = 2
NS = 16
L = 16
BLK = 64
EPAD = 40960
NBLK = EPAD // (NS * BLK)
NG = 4
NGRP = NBLK // NG
DH = 128
NROW = 10240
RPS = NROW // NS
ZB = 128

_sc_mesh = plsc.VectorSubcoreMesh(
    core_axis_name="c", subcore_axis_name="s", num_cores=NC, num_subcores=NS)


def _fill(ref, col):
  for j in range(DH // L):
    vals = jnp.where(lax.iota(jnp.int32, L) + j * L == col,
                     jnp.float32(1.0), jnp.float32(0.0))

    def body(i, _, j=j, vals=vals):
      ref[i, pl.ds(j * L, L)] = vals
      return 0

    lax.fori_loop(0, ZB, body, 0)


def _make_sc_kernel():
  @functools.partial(
      pl.kernel,
      out_type=(
          jax.ShapeDtypeStruct((NC, R, NROW, DH), jnp.float32),
          jax.ShapeDtypeStruct((NC, NROW, DH), jnp.float32),
      ),
      mesh=_sc_mesh,
      scratch_types=[
          pltpu.VMEM((ZB, DH), jnp.float32),
          pltpu.VMEM((NG, BLK), jnp.int32),
          pltpu.VMEM((NG, BLK), jnp.int32),
          pltpu.VMEM((NG, BLK), jnp.int32),
          pltpu.VMEM((NG, BLK), jnp.int32),
          pltpu.VMEM((BLK, DH), jnp.float32),
          pltpu.VMEM((BLK, DH), jnp.float32),
          pltpu.VMEM_SHARED((NROW, DH), jnp.float32),
          pltpu.SemaphoreType.DMA,
          pltpu.SemaphoreType.DMA,
          pltpu.SemaphoreType.DMA,
          pltpu.SemaphoreType.DMA,
      ],
  )
  def sc_agg(xs_hbm, ei_hbm, agg_hbm, deg_hbm,
             fbuf, srcbA, srcbB, dstbA, dstbB, rowb0, rowb1, shared,
             sem0, sem1, semiA, semiB):
    c = lax.axis_index("c")
    s = lax.axis_index("s")
    sems = (sem0, sem1)
    rowbs = (rowb0, rowb1)
    srcbs = (srcbA, srcbB)
    dstbs = (dstbA, dstbB)
    semis = (semiA, semiB)

    def idx_prefetch(r, g, p):
      return (pltpu.async_copy(ei_hbm.at[r, 0, s, pl.ds(g * NG, NG)],
                               srcbs[p], semis[p]),
              pltpu.async_copy(ei_hbm.at[r, 1, s, pl.ds(g * NG, NG)],
                               dstbs[p], semis[p]))

    def zero_my_rows():
      for z in range(RPS // ZB):
        pltpu.sync_copy(fbuf, shared.at[pl.ds(s * RPS + z * ZB, ZB)])

    _fill(fbuf, jnp.int32(-1))

    for r in range(R):
      zero_my_rows()
      plsc.subcore_barrier()

      dA = idx_prefetch(r, 0, 0)
      for g in range(NGRP):
        p = g % 2
        dB = idx_prefetch(r, g + 1, 1 - p) if g + 1 < NGRP else None
        dA[0].wait()
        dA[1].wait()
        srcb, dstb = srcbs[p], dstbs[p]
        d = pltpu.async_copy(xs_hbm.at[c].at[srcb.at[0]], rowbs[0], sems[0])
        for j in range(NG):
          nxt = None
          if j + 1 < NG:
            nxt = pltpu.async_copy(
                xs_hbm.at[c].at[srcb.at[j + 1]],
                rowbs[(j + 1) % 2], sems[(j + 1) % 2])
          d.wait()
          pltpu.sync_copy(rowbs[j % 2], shared.at[dstb.at[j]], add=True)
          d = nxt
        dA = dB
      plsc.subcore_barrier()

      for z in range(RPS // ZB):
        rowlo = s * RPS + z * ZB
        pltpu.sync_copy(shared.at[pl.ds(rowlo, ZB)],
                        agg_hbm.at[c, r, pl.ds(rowlo, ZB)])

    zero_my_rows()
    plsc.subcore_barrier()
    for rr in range(NC):
      r = NC * c + rr
      _fill(fbuf, r)
      dA = pltpu.async_copy(ei_hbm.at[r, 1, s, pl.ds(0, NG)],
                            dstbs[0], semis[0])
      for g in range(NGRP):
        p = g % 2
        dB = (pltpu.async_copy(ei_hbm.at[r, 1, s, pl.ds((g + 1) * NG, NG)],
                               dstbs[1 - p], semis[1 - p])
              if g + 1 < NGRP else None)
        dA.wait()
        for j in range(NG):
          pltpu.sync_copy(fbuf.at[pl.ds(0, BLK)],
                          shared.at[dstbs[p].at[j]], add=True)
        dA = dB
      plsc.subcore_barrier()
    for z in range(RPS // ZB):
      rowlo = s * RPS + z * ZB
      pltpu.sync_copy(shared.at[pl.ds(rowlo, ZB)],
                      deg_hbm.at[c, pl.ds(rowlo, ZB)])

  return sc_agg


_sc_agg = _make_sc_kernel()

MB = 2000


def _tc_body(x_ref, agg_ref, deg_ref, w_ref, lw_ref, o_ref):
  acc = jnp.dot(x_ref[...], lw_ref[...], preferred_element_type=jnp.float32)
  for r in range(R):
    deg = deg_ref[r // NC, :, r:r + 1]
    rec = 1.0 / jnp.maximum(deg, 1.0)
    acc = acc + jnp.dot(agg_ref[0, r] * rec, w_ref[r, 0:DH, :],
                        preferred_element_type=jnp.float32)
    acc = acc + jnp.dot(agg_ref[1, r] * rec, w_ref[r, DH:D, :],
                        preferred_element_type=jnp.float32)
  o_ref[...] = acc


def _tc_matmul(x, agg, deg, w, lw):
  return pl.pallas_call(
      _tc_body,
      grid=(N // MB,),
      in_specs=[
          pl.BlockSpec((MB, D), lambda i: (i, 0)),
          pl.BlockSpec((NC, R, MB, DH), lambda i: (0, 0, i, 0)),
          pl.BlockSpec((NC, MB, DH), lambda i: (0, i, 0)),
          pl.BlockSpec((R, D, D), lambda i: (0, 0, 0)),
          pl.BlockSpec((D, D), lambda i: (0, 0)),
      ],
      out_specs=pl.BlockSpec((MB, D), lambda i: (i, 0)),
      out_shape=jax.ShapeDtypeStruct((N, D), jnp.float32),
  )(x, agg, deg, w, lw)


def kernel(x, edge_index_r0, edge_index_r1, edge_index_r2, edge_index_r3,
           weight, loop_weight):
  xs = x.reshape(N, NC, DH).transpose(1, 0, 2)

  pad = EPAD - E
  eis = []
  for ei in (edge_index_r0, edge_index_r1, edge_index_r2, edge_index_r3):
    eis.append(jnp.concatenate(
        [ei, jnp.concatenate([jnp.zeros((1, pad), jnp.int32),
                              jnp.full((1, pad), N, jnp.int32)], axis=0)],
        axis=1))
  ei = jnp.stack(eis).reshape(R, 2, NS, NBLK, BLK)

  agg, deg = _sc_agg(xs, ei)
  return _tc_matmul(x, agg, deg, weight, loop_weight)
